# baseline (speedup 1.0000x reference)
import jax
import jax.numpy as jnp
import numpy as np
from jax import lax
from jax.experimental import pallas as pl
from jax.experimental.pallas import tpu as pltpu
from jax.experimental.pallas import tpu_sc as plsc

_N = 10000
_E = 320000
_H = 128
_B = 64
_P = 5

_NC = 2
_NS = 16
_NW = _NC * _NS

_CH = 128
_NCHUNK = _E // _CH
_BASE_CHUNKS = _NCHUNK // _NW
_EXTRA = _NCHUNK - _BASE_CHUNKS * _NW
_NB = 80
_EP = _NW * _NB * _CH
_NROW = _NW * _NB
_SLOTS = 4
_NG = _NB // _SLOTS
_GS = 2
_PNB = 40
_NPH = _NB // _PNB
_PNG = _PNB // _GS

_NP = 10240
_RPS = _NP // _NS

_NREAL = _E // _CH
def _make_perm():
    perm = []
    pad_ptr = _NREAL
    for w in range(_NW):
        nb = 79 if w < 4 else 78
        perm.extend(w + _NW * j for j in range(nb))
        perm.extend(range(pad_ptr, pad_ptr + _NB - nb))
        pad_ptr += _NB - nb
    return np.asarray(perm, dtype=np.int32)
_PERM = _make_perm()


def _sc_mesh():
    return plsc.VectorSubcoreMesh(
        core_axis_name="c", subcore_axis_name="s",
        num_cores=_NC, num_subcores=_NS)


_BLK = 4
_NBLK = _NB // _BLK


def _seg_rows_body(x_hbm, ed_hbm, zr_hbm, out_hbm,
                   idxb, rows0, rows1, agg, sg, sw0, sw1):
    c = lax.axis_index("c")
    s = lax.axis_index("s")
    wid = s * _NC + c
    rows = (rows0, rows1)
    sw = (sw0, sw1)

    pltpu.sync_copy(zr_hbm, agg.at[pl.ds(s * _RPS, _RPS)])
    plsc.subcore_barrier()

    def load_idx(t):
        pltpu.sync_copy(ed_hbm.at[pl.ds((wid * _NBLK + t) * 2 * _BLK,
                                        2 * _BLK)], idxb)

    def block(t, _):
        load_idx(t)
        for k in range(_BLK):
            pltpu.async_copy(x_hbm.at[idxb.at[k]], rows0, sg).wait()
            pltpu.sync_copy(rows0, agg.at[idxb.at[_BLK + k]], add=True)
        return 0
    lax.fori_loop(0, _NBLK, block, 0)

    plsc.subcore_barrier()
    pltpu.sync_copy(agg.at[pl.ds(s * _RPS, _RPS)],
                    out_hbm.at[pl.ds(c * _NP + s * _RPS, _RPS)])


def _sc_segment_rows(x, ed, zrows):
    return pl.kernel(
        _seg_rows_body,
        out_type=jax.ShapeDtypeStruct((_NC * _NP, _H), jnp.float32),
        mesh=_sc_mesh(),
        scratch_types=[
            pltpu.VMEM((2 * _BLK, _CH), jnp.int32),
            pltpu.VMEM((_CH, _H), jnp.float32),
            pltpu.VMEM((_CH, _H), jnp.float32),
            pltpu.VMEM_SHARED((_NP, _H), jnp.float32),
        ] + [pltpu.SemaphoreType.DMA] * 3,
    )(x, ed, zrows)


def _deg_body(dst_hbm, ones_hbm, zr_hbm, out_hbm, idx_da, ones_v, agg, *sems):
    c = lax.axis_index("c")
    s = lax.axis_index("s")
    wid = s * _NC + c
    sw = sems

    pltpu.sync_copy(zr_hbm, agg.at[pl.ds(s * _RPS, _RPS)])
    pltpu.sync_copy(ones_hbm, ones_v)
    pltpu.sync_copy(dst_hbm.at[pl.ds(wid * _NB, _NB)], idx_da)
    plsc.subcore_barrier()

    def scatter(j, slot):
        pltpu.async_copy(ones_v, agg.at[idx_da.at[j]], sw[slot], add=True)

    def wait_scatter(slot):
        pltpu.make_async_copy(
            ones_v, agg.at[idx_da.at[0]], sw[slot]).wait()

    for slot in range(_SLOTS):
        scatter(slot, slot)

    def group(g, _):
        base = (g + 1) * _SLOTS
        for slot in range(_SLOTS):
            wait_scatter(slot)
            scatter(base + slot, slot)
        return 0
    lax.fori_loop(0, _NG - 1, group, 0)
    for slot in range(_SLOTS):
        wait_scatter(slot)

    plsc.subcore_barrier()
    pltpu.sync_copy(agg.at[pl.ds(s * _RPS, _RPS)],
                    out_hbm.at[pl.ds(c * _NP + s * _RPS, _RPS)])


def _sc_degrees(dst2d, ones, zrows):
    return pl.kernel(
        _deg_body,
        out_type=jax.ShapeDtypeStruct((_NC * _NP, _H), jnp.float32),
        mesh=_sc_mesh(),
        scratch_types=[
            pltpu.VMEM((_NB, _CH), jnp.int32),
            pltpu.VMEM((_CH, _H), jnp.float32),
            pltpu.VMEM_SHARED((_NP, _H), jnp.float32),
        ] + [pltpu.SemaphoreType.DMA] * _SLOTS,
    )(dst2d, ones, zrows)


def _embed_body(h_ref, w_ref, b_ref, o_ref):
    o_ref[...] = lax.dot_general(
        h_ref[...], w_ref[...], (((1,), (0,)), ((), ())),
        preferred_element_type=jnp.float32) + b_ref[...]


def _tc_embed(h, W_emb, b_emb2d):
    return pl.pallas_call(
        _embed_body,
        out_shape=jax.ShapeDtypeStruct((_N, _H), jnp.float32),
    )(h, W_emb, b_emb2d)


def _layer_body(x_ref, parts_ref, degp_ref, ws_ref, bs_ref, wn_ref, bn_ref,
                g_ref, bt_ref, o_ref):
    x = x_ref[...]
    deg = degp_ref[0:_N, 0:1] + degp_ref[_NP:_NP + _N, 0:1]
    rdeg = 1.0 / jnp.maximum(deg, 1.0)
    agg = (parts_ref[0:_N, :] + parts_ref[_NP:_NP + _N, :]) * rdeg
    out = (lax.dot_general(x, ws_ref[...], (((1,), (0,)), ((), ())),
                           preferred_element_type=jnp.float32)
           + bs_ref[...]
           + lax.dot_general(agg, wn_ref[...], (((1,), (0,)), ((), ())),
                             preferred_element_type=jnp.float32)
           + bn_ref[...])
    out = jnp.maximum(out, 0.0)
    mu = jnp.mean(out, axis=0, keepdims=True)
    var = jnp.mean((out - mu) ** 2, axis=0, keepdims=True)
    out = g_ref[...] * (out - mu) / jnp.sqrt(var + 1e-5) + bt_ref[...]
    o_ref[...] = x + out


def _tc_layer(x, parts, degp, Ws, bs2d, Wn, bn2d, g2d, bt2d):
    return pl.pallas_call(
        _layer_body,
        out_shape=jax.ShapeDtypeStruct((_N, _H), jnp.float32),
    )(x, parts, degp, Ws, bs2d, Wn, bn2d, g2d, bt2d)


def _head_body(x_ref, gid_ref, pp_ref, pn_ref, wfc_ref, o_ref):
    x = x_ref[...]
    ids = gid_ref[...]
    iota = lax.broadcasted_iota(jnp.int32, (_N, _B), 1)
    mask = (ids == iota).astype(jnp.float32)
    cnt = jnp.sum(mask, axis=0, keepdims=True)
    hgs = lax.dot_general(mask, x, (((0,), (0,)), ((), ())),
                          preferred_element_type=jnp.float32)
    hg = hgs / jnp.maximum(cnt, 1.0).reshape(_B, 1)
    cols = []
    for i in range(_P):
        dp = jnp.sum((hg - pp_ref[i:i + 1, :]) ** 2, axis=1, keepdims=True)
        cols.append(dp)
    for i in range(_P):
        dn = jnp.sum((hg - pn_ref[i:i + 1, :]) ** 2, axis=1, keepdims=True)
        cols.append(dn)
    d = jnp.concatenate(cols, axis=1)
    ss = jnp.log((d + 1.0) / (d + 1e-12))
    y = lax.dot_general(ss, wfc_ref[...], (((1,), (1,)), ((), ())),
                        preferred_element_type=jnp.float32)
    o_ref[...] = 1.0 / (1.0 + jnp.exp(-y))


def _tc_head(x, gid2d, p_pos, p_neg, W_fc):
    return pl.pallas_call(
        _head_body,
        out_shape=jax.ShapeDtypeStruct((_B, 2), jnp.float32),
    )(x, gid2d, p_pos, p_neg, W_fc)


@jax.jit
def kernel(h, edge_index, e, graph_ids, W_emb, b_emb, W_self, b_self,
           W_neigh, b_neigh, gamma, beta, p_pos, p_neg, W_fc):
    pad = _EP - _E
    src2d_pad = jnp.concatenate(
        [edge_index[0], jnp.zeros((pad,), jnp.int32)]).reshape(_NROW, _CH)
    dst2d_pad = jnp.concatenate(
        [edge_index[1],
         _N + (jnp.arange(pad, dtype=jnp.int32) % (_NP - _N))]
    ).reshape(_NROW, _CH)
    ed = jnp.concatenate(
        [src2d_pad.reshape(_NROW // _BLK, _BLK, _CH),
         dst2d_pad.reshape(_NROW // _BLK, _BLK, _CH)],
        axis=1).reshape(2 * _NROW, _CH)
    gid2d = graph_ids.reshape(_N, 1)
    zrows = jnp.zeros((_RPS, _H), jnp.float32)
    ones = jnp.ones((_CH, _H), jnp.float32)

    x = _tc_embed(h, W_emb, b_emb.reshape(1, _H))
    degp = _sc_degrees(dst2d_pad, ones, zrows)
    for l in range(3):
        parts = _sc_segment_rows(x, ed, zrows)
        x = _tc_layer(x, parts, degp,
                      W_self[l], b_self[l].reshape(1, _H),
                      W_neigh[l], b_neigh[l].reshape(1, _H),
                      gamma[l].reshape(1, _H), beta[l].reshape(1, _H))
    return _tc_head(x, gid2d, p_pos, p_neg, W_fc)


# speedup vs baseline: 2.1573x; 2.1573x over previous
import jax
import jax.numpy as jnp
import numpy as np
from jax import lax
from jax.experimental import pallas as pl
from jax.experimental.pallas import tpu as pltpu
from jax.experimental.pallas import tpu_sc as plsc

_N = 10000
_E = 320000
_H = 128
_B = 64
_P = 5

_NC = 2
_NS = 16
_NW = _NC * _NS

_CH = 128
_NCHUNK = _E // _CH
_BASE_CHUNKS = _NCHUNK // _NW
_EXTRA = _NCHUNK - _BASE_CHUNKS * _NW
_NB = 80
_EP = _NW * _NB * _CH
_NROW = _NW * _NB
_SLOTS = 4
_NG = _NB // _SLOTS
_GS = 2
_PNB = 40
_NPH = _NB // _PNB
_PNG = _PNB // _GS

_NP = 10240
_RPS = _NP // _NS

_NREAL = _E // _CH
def _make_perm():
    perm = []
    pad_ptr = _NREAL
    for w in range(_NW):
        nb = 79 if w < 4 else 78
        perm.extend(w + _NW * j for j in range(nb))
        perm.extend(range(pad_ptr, pad_ptr + _NB - nb))
        pad_ptr += _NB - nb
    return np.asarray(perm, dtype=np.int32)
_PERM = _make_perm()


def _sc_mesh():
    return plsc.VectorSubcoreMesh(
        core_axis_name="c", subcore_axis_name="s",
        num_cores=_NC, num_subcores=_NS)


def _seg_rows_body(x_hbm, src_hbm, dst_hbm, zr_hbm, out_hbm,
                   idx_s, idx_d, rows, agg, sg):
    c = lax.axis_index("c")
    s = lax.axis_index("s")
    wid = s * _NC + c

    pltpu.sync_copy(zr_hbm, agg.at[pl.ds(s * _RPS, _RPS)])
    plsc.subcore_barrier()

    def step(j, _):
        q = wid + j * _NW
        pltpu.sync_copy(src_hbm.at[q], idx_s)
        pltpu.sync_copy(dst_hbm.at[q], idx_d.at[0])
        pltpu.async_copy(x_hbm.at[idx_s], rows, sg).wait()
        pltpu.sync_copy(rows, agg.at[idx_d.at[0]], add=True)
        return 0
    lax.fori_loop(0, _NB, step, 0)

    plsc.subcore_barrier()
    pltpu.sync_copy(agg.at[pl.ds(s * _RPS, _RPS)],
                    out_hbm.at[pl.ds(c * _NP + s * _RPS, _RPS)])


def _sc_segment_rows(x, src2d, dst2d, zrows):
    return pl.kernel(
        _seg_rows_body,
        out_type=jax.ShapeDtypeStruct((_NC * _NP, _H), jnp.float32),
        mesh=_sc_mesh(),
        scratch_types=[
            pltpu.VMEM((_CH,), jnp.int32),
            pltpu.VMEM((1, _CH), jnp.int32),
            pltpu.VMEM((_CH, _H), jnp.float32),
            pltpu.VMEM_SHARED((_NP, _H), jnp.float32),
            pltpu.SemaphoreType.DMA,
        ],
    )(x, src2d, dst2d, zrows)


def _deg_body(dst_hbm, ones_hbm, zr_hbm, out_hbm, idx_da, ones_v, agg, *sems):
    c = lax.axis_index("c")
    s = lax.axis_index("s")
    wid = s * _NC + c
    sw = sems

    pltpu.sync_copy(zr_hbm, agg.at[pl.ds(s * _RPS, _RPS)])
    pltpu.sync_copy(ones_hbm, ones_v)
    pltpu.sync_copy(dst_hbm.at[pl.ds(wid * _NB, _NB)], idx_da)
    plsc.subcore_barrier()

    def scatter(j, slot):
        pltpu.async_copy(ones_v, agg.at[idx_da.at[j]], sw[slot], add=True)

    def wait_scatter(slot):
        pltpu.make_async_copy(
            ones_v, agg.at[idx_da.at[0]], sw[slot]).wait()

    for slot in range(_SLOTS):
        scatter(slot, slot)

    def group(g, _):
        base = (g + 1) * _SLOTS
        for slot in range(_SLOTS):
            wait_scatter(slot)
            scatter(base + slot, slot)
        return 0
    lax.fori_loop(0, _NG - 1, group, 0)
    for slot in range(_SLOTS):
        wait_scatter(slot)

    plsc.subcore_barrier()
    pltpu.sync_copy(agg.at[pl.ds(s * _RPS, _RPS)],
                    out_hbm.at[pl.ds(c * _NP + s * _RPS, _RPS)])


def _sc_degrees(dst2d, ones, zrows):
    return pl.kernel(
        _deg_body,
        out_type=jax.ShapeDtypeStruct((_NC * _NP, _H), jnp.float32),
        mesh=_sc_mesh(),
        scratch_types=[
            pltpu.VMEM((_NB, _CH), jnp.int32),
            pltpu.VMEM((_CH, _H), jnp.float32),
            pltpu.VMEM_SHARED((_NP, _H), jnp.float32),
        ] + [pltpu.SemaphoreType.DMA] * _SLOTS,
    )(dst2d, ones, zrows)


def _embed_body(h_ref, w_ref, b_ref, o_ref):
    o_ref[...] = lax.dot_general(
        h_ref[...], w_ref[...], (((1,), (0,)), ((), ())),
        preferred_element_type=jnp.float32) + b_ref[...]


def _tc_embed(h, W_emb, b_emb2d):
    return pl.pallas_call(
        _embed_body,
        out_shape=jax.ShapeDtypeStruct((_N, _H), jnp.float32),
    )(h, W_emb, b_emb2d)


def _layer_body(x_ref, parts_ref, degp_ref, ws_ref, bs_ref, wn_ref, bn_ref,
                g_ref, bt_ref, o_ref):
    x = x_ref[...]
    deg = degp_ref[0:_N, 0:1] + degp_ref[_NP:_NP + _N, 0:1]
    rdeg = 1.0 / jnp.maximum(deg, 1.0)
    agg = (parts_ref[0:_N, :] + parts_ref[_NP:_NP + _N, :]) * rdeg
    out = (lax.dot_general(x, ws_ref[...], (((1,), (0,)), ((), ())),
                           preferred_element_type=jnp.float32)
           + bs_ref[...]
           + lax.dot_general(agg, wn_ref[...], (((1,), (0,)), ((), ())),
                             preferred_element_type=jnp.float32)
           + bn_ref[...])
    out = jnp.maximum(out, 0.0)
    mu = jnp.mean(out, axis=0, keepdims=True)
    var = jnp.mean((out - mu) ** 2, axis=0, keepdims=True)
    out = g_ref[...] * (out - mu) / jnp.sqrt(var + 1e-5) + bt_ref[...]
    o_ref[...] = x + out


def _tc_layer(x, parts, degp, Ws, bs2d, Wn, bn2d, g2d, bt2d):
    return pl.pallas_call(
        _layer_body,
        out_shape=jax.ShapeDtypeStruct((_N, _H), jnp.float32),
    )(x, parts, degp, Ws, bs2d, Wn, bn2d, g2d, bt2d)


def _head_body(x_ref, gid_ref, pp_ref, pn_ref, wfc_ref, o_ref):
    x = x_ref[...]
    ids = gid_ref[...]
    iota = lax.broadcasted_iota(jnp.int32, (_N, _B), 1)
    mask = (ids == iota).astype(jnp.float32)
    cnt = jnp.sum(mask, axis=0, keepdims=True)
    hgs = lax.dot_general(mask, x, (((0,), (0,)), ((), ())),
                          preferred_element_type=jnp.float32)
    hg = hgs / jnp.maximum(cnt, 1.0).reshape(_B, 1)
    cols = []
    for i in range(_P):
        dp = jnp.sum((hg - pp_ref[i:i + 1, :]) ** 2, axis=1, keepdims=True)
        cols.append(dp)
    for i in range(_P):
        dn = jnp.sum((hg - pn_ref[i:i + 1, :]) ** 2, axis=1, keepdims=True)
        cols.append(dn)
    d = jnp.concatenate(cols, axis=1)
    ss = jnp.log((d + 1.0) / (d + 1e-12))
    y = lax.dot_general(ss, wfc_ref[...], (((1,), (1,)), ((), ())),
                        preferred_element_type=jnp.float32)
    o_ref[...] = 1.0 / (1.0 + jnp.exp(-y))


def _tc_head(x, gid2d, p_pos, p_neg, W_fc):
    return pl.pallas_call(
        _head_body,
        out_shape=jax.ShapeDtypeStruct((_B, 2), jnp.float32),
    )(x, gid2d, p_pos, p_neg, W_fc)


@jax.jit
def kernel(h, edge_index, e, graph_ids, W_emb, b_emb, W_self, b_self,
           W_neigh, b_neigh, gamma, beta, p_pos, p_neg, W_fc):
    pad = _EP - _E
    src2d_pad = jnp.concatenate(
        [edge_index[0], jnp.arange(pad, dtype=jnp.int32) % _N]
    ).reshape(_NROW, _CH)
    dst2d_pad = jnp.concatenate(
        [edge_index[1],
         _N + (jnp.arange(pad, dtype=jnp.int32) % (_NP - _N))]
    ).reshape(_NROW, _CH)
    gid2d = graph_ids.reshape(_N, 1)
    zrows = jnp.zeros((_RPS, _H), jnp.float32)
    ones = jnp.ones((_CH, _H), jnp.float32)

    x = _tc_embed(h, W_emb, b_emb.reshape(1, _H))
    degp = _sc_degrees(dst2d_pad, ones, zrows)
    for l in range(3):
        parts = _sc_segment_rows(x, src2d_pad, dst2d_pad, zrows)
        x = _tc_layer(x, parts, degp,
                      W_self[l], b_self[l].reshape(1, _H),
                      W_neigh[l], b_neigh[l].reshape(1, _H),
                      gamma[l].reshape(1, _H), beta[l].reshape(1, _H))
    return _tc_head(x, gid2d, p_pos, p_neg, W_fc)


# speedup vs baseline: 2.6596x; 1.2328x over previous
import jax
import jax.numpy as jnp
import numpy as np
from jax import lax
from jax.experimental import pallas as pl
from jax.experimental.pallas import tpu as pltpu
from jax.experimental.pallas import tpu_sc as plsc

_N = 10000
_E = 320000
_H = 128
_B = 64
_P = 5

_NC = 2
_NS = 16
_NW = _NC * _NS

_CH = 128
_NCHUNK = _E // _CH
_BASE_CHUNKS = _NCHUNK // _NW
_EXTRA = _NCHUNK - _BASE_CHUNKS * _NW
_NB = 80
_EP = _NW * _NB * _CH
_NROW = _NW * _NB
_SLOTS = 4
_NG = _NB // _SLOTS
_GS = 2
_PNB = 40
_NPH = _NB // _PNB
_PNG = _PNB // _GS

_NP = 10240
_RPS = _NP // _NS

_NREAL = _E // _CH
def _make_edq():
    q = np.empty((_NROW,), dtype=np.int32)
    for w in range(_NW):
        for j in range(_NB):
            q[w * _NB + j] = w + _NW * j
    return q
_EDQ = _make_edq()


def _sc_mesh():
    return plsc.VectorSubcoreMesh(
        core_axis_name="c", subcore_axis_name="s",
        num_cores=_NC, num_subcores=_NS)


_BLK = 4
_NBLK = _NB // _BLK


def _seg_rows_body(x_hbm, ed_hbm, zr_hbm, out_hbm, idxb, rows, agg, sg):
    c = lax.axis_index("c")
    s = lax.axis_index("s")
    wid = s * _NC + c

    pltpu.sync_copy(zr_hbm, agg.at[pl.ds(s * _RPS, _RPS)])
    plsc.subcore_barrier()

    def block(t, _):
        pltpu.sync_copy(
            ed_hbm.at[pl.ds((wid * _NBLK + t) * 2 * _BLK, 2 * _BLK)], idxb)
        for k in range(_BLK):
            pltpu.async_copy(x_hbm.at[idxb.at[k]], rows, sg).wait()
            pltpu.sync_copy(rows, agg.at[idxb.at[_BLK + k]], add=True)
        return 0
    lax.fori_loop(0, _NBLK, block, 0)

    plsc.subcore_barrier()
    pltpu.sync_copy(agg.at[pl.ds(s * _RPS, _RPS)],
                    out_hbm.at[pl.ds(c * _NP + s * _RPS, _RPS)])


def _sc_segment_rows(x, ed, zrows):
    return pl.kernel(
        _seg_rows_body,
        out_type=jax.ShapeDtypeStruct((_NC * _NP, _H), jnp.float32),
        mesh=_sc_mesh(),
        scratch_types=[
            pltpu.VMEM((2 * _BLK, _CH), jnp.int32),
            pltpu.VMEM((_CH, _H), jnp.float32),
            pltpu.VMEM_SHARED((_NP, _H), jnp.float32),
            pltpu.SemaphoreType.DMA,
        ],
    )(x, ed, zrows)


def _deg_body(dst_hbm, ones_hbm, zr_hbm, out_hbm, idx_da, ones_v, agg, *sems):
    c = lax.axis_index("c")
    s = lax.axis_index("s")
    wid = s * _NC + c
    sw = sems

    pltpu.sync_copy(zr_hbm, agg.at[pl.ds(s * _RPS, _RPS)])
    pltpu.sync_copy(ones_hbm, ones_v)
    pltpu.sync_copy(dst_hbm.at[pl.ds(wid * _NB, _NB)], idx_da)
    plsc.subcore_barrier()

    def scatter(j, slot):
        pltpu.async_copy(ones_v, agg.at[idx_da.at[j]], sw[slot], add=True)

    def wait_scatter(slot):
        pltpu.make_async_copy(
            ones_v, agg.at[idx_da.at[0]], sw[slot]).wait()

    for slot in range(_SLOTS):
        scatter(slot, slot)

    def group(g, _):
        base = (g + 1) * _SLOTS
        for slot in range(_SLOTS):
            wait_scatter(slot)
            scatter(base + slot, slot)
        return 0
    lax.fori_loop(0, _NG - 1, group, 0)
    for slot in range(_SLOTS):
        wait_scatter(slot)

    plsc.subcore_barrier()
    pltpu.sync_copy(agg.at[pl.ds(s * _RPS, _RPS)],
                    out_hbm.at[pl.ds(c * _NP + s * _RPS, _RPS)])


def _sc_degrees(dst2d, ones, zrows):
    return pl.kernel(
        _deg_body,
        out_type=jax.ShapeDtypeStruct((_NC * _NP, _H), jnp.float32),
        mesh=_sc_mesh(),
        scratch_types=[
            pltpu.VMEM((_NB, _CH), jnp.int32),
            pltpu.VMEM((_CH, _H), jnp.float32),
            pltpu.VMEM_SHARED((_NP, _H), jnp.float32),
        ] + [pltpu.SemaphoreType.DMA] * _SLOTS,
    )(dst2d, ones, zrows)


def _embed_body(h_ref, w_ref, b_ref, o_ref):
    o_ref[...] = lax.dot_general(
        h_ref[...], w_ref[...], (((1,), (0,)), ((), ())),
        preferred_element_type=jnp.float32) + b_ref[...]


def _tc_embed(h, W_emb, b_emb2d):
    return pl.pallas_call(
        _embed_body,
        out_shape=jax.ShapeDtypeStruct((_N, _H), jnp.float32),
    )(h, W_emb, b_emb2d)


def _layer_body(x_ref, parts_ref, degp_ref, ws_ref, bs_ref, wn_ref, bn_ref,
                g_ref, bt_ref, o_ref):
    x = x_ref[...]
    deg = degp_ref[0:_N, 0:1] + degp_ref[_NP:_NP + _N, 0:1]
    rdeg = 1.0 / jnp.maximum(deg, 1.0)
    agg = (parts_ref[0:_N, :] + parts_ref[_NP:_NP + _N, :]) * rdeg
    out = (lax.dot_general(x, ws_ref[...], (((1,), (0,)), ((), ())),
                           preferred_element_type=jnp.float32)
           + bs_ref[...]
           + lax.dot_general(agg, wn_ref[...], (((1,), (0,)), ((), ())),
                             preferred_element_type=jnp.float32)
           + bn_ref[...])
    out = jnp.maximum(out, 0.0)
    mu = jnp.mean(out, axis=0, keepdims=True)
    var = jnp.mean((out - mu) ** 2, axis=0, keepdims=True)
    out = g_ref[...] * (out - mu) / jnp.sqrt(var + 1e-5) + bt_ref[...]
    o_ref[...] = x + out


def _tc_layer(x, parts, degp, Ws, bs2d, Wn, bn2d, g2d, bt2d):
    return pl.pallas_call(
        _layer_body,
        out_shape=jax.ShapeDtypeStruct((_N, _H), jnp.float32),
    )(x, parts, degp, Ws, bs2d, Wn, bn2d, g2d, bt2d)


def _head_body(x_ref, gid_ref, pp_ref, pn_ref, wfc_ref, o_ref):
    x = x_ref[...]
    ids = gid_ref[...]
    iota = lax.broadcasted_iota(jnp.int32, (_N, _B), 1)
    mask = (ids == iota).astype(jnp.float32)
    cnt = jnp.sum(mask, axis=0, keepdims=True)
    hgs = lax.dot_general(mask, x, (((0,), (0,)), ((), ())),
                          preferred_element_type=jnp.float32)
    hg = hgs / jnp.maximum(cnt, 1.0).reshape(_B, 1)
    cols = []
    for i in range(_P):
        dp = jnp.sum((hg - pp_ref[i:i + 1, :]) ** 2, axis=1, keepdims=True)
        cols.append(dp)
    for i in range(_P):
        dn = jnp.sum((hg - pn_ref[i:i + 1, :]) ** 2, axis=1, keepdims=True)
        cols.append(dn)
    d = jnp.concatenate(cols, axis=1)
    ss = jnp.log((d + 1.0) / (d + 1e-12))
    y = lax.dot_general(ss, wfc_ref[...], (((1,), (1,)), ((), ())),
                        preferred_element_type=jnp.float32)
    o_ref[...] = 1.0 / (1.0 + jnp.exp(-y))


def _tc_head(x, gid2d, p_pos, p_neg, W_fc):
    return pl.pallas_call(
        _head_body,
        out_shape=jax.ShapeDtypeStruct((_B, 2), jnp.float32),
    )(x, gid2d, p_pos, p_neg, W_fc)


@jax.jit
def kernel(h, edge_index, e, graph_ids, W_emb, b_emb, W_self, b_self,
           W_neigh, b_neigh, gamma, beta, p_pos, p_neg, W_fc):
    pad = _EP - _E
    src2d_pad = jnp.concatenate(
        [edge_index[0], jnp.arange(pad, dtype=jnp.int32) % _N]
    ).reshape(_NROW, _CH)
    dst2d_pad = jnp.concatenate(
        [edge_index[1],
         _N + (jnp.arange(pad, dtype=jnp.int32) % (_NP - _N))]
    ).reshape(_NROW, _CH)
    q = jnp.asarray(_EDQ)
    ed = jnp.concatenate(
        [src2d_pad[q].reshape(-1, _BLK, _CH),
         dst2d_pad[q].reshape(-1, _BLK, _CH)],
        axis=1).reshape(2 * _NROW, _CH)
    gid2d = graph_ids.reshape(_N, 1)
    zrows = jnp.zeros((_RPS, _H), jnp.float32)
    ones = jnp.ones((_CH, _H), jnp.float32)

    x = _tc_embed(h, W_emb, b_emb.reshape(1, _H))
    degp = _sc_degrees(dst2d_pad, ones, zrows)
    for l in range(3):
        parts = _sc_segment_rows(x, ed, zrows)
        x = _tc_layer(x, parts, degp,
                      W_self[l], b_self[l].reshape(1, _H),
                      W_neigh[l], b_neigh[l].reshape(1, _H),
                      gamma[l].reshape(1, _H), beta[l].reshape(1, _H))
    return _tc_head(x, gid2d, p_pos, p_neg, W_fc)


# speedup vs baseline: 3.1016x; 1.1662x over previous
import jax
import jax.numpy as jnp
import numpy as np
from jax import lax
from jax.experimental import pallas as pl
from jax.experimental.pallas import tpu as pltpu
from jax.experimental.pallas import tpu_sc as plsc

_N = 10000
_E = 320000
_H = 128
_B = 64
_P = 5

_NC = 2
_NS = 16
_NW = _NC * _NS

_CH = 128
_NCHUNK = _E // _CH
_BASE_CHUNKS = _NCHUNK // _NW
_EXTRA = _NCHUNK - _BASE_CHUNKS * _NW
_NB = 80
_EP = _NW * _NB * _CH
_NROW = _NW * _NB
_SLOTS = 4
_NG = _NB // _SLOTS
_GS = 2
_PNB = 40
_NPH = _NB // _PNB
_PNG = _PNB // _GS

_NP = 10240
_RPS = _NP // _NS

_NREAL = _E // _CH
def _make_edq():
    q = np.empty((_NROW,), dtype=np.int32)
    for w in range(_NW):
        for j in range(_NB):
            q[w * _NB + j] = w + _NW * j
    return q
_EDQ = _make_edq()


def _sc_mesh():
    return plsc.VectorSubcoreMesh(
        core_axis_name="c", subcore_axis_name="s",
        num_cores=_NC, num_subcores=_NS)


_BLK = 4
_NBLK = _NB // _BLK


def _seg_rows_body(x_hbm, ed_hbm, zr_hbm, out_hbm,
                   idxb, rows0, rows1, agg, sg, sw0, sw1):
    c = lax.axis_index("c")
    s = lax.axis_index("s")
    wid = s * _NC + c
    rows = (rows0, rows1)
    sw = (sw0, sw1)

    pltpu.sync_copy(zr_hbm, agg.at[pl.ds(s * _RPS, _RPS)])
    plsc.subcore_barrier()

    def load_idx(t):
        pltpu.sync_copy(
            ed_hbm.at[pl.ds((wid * _NBLK + t) * 2 * _BLK, 2 * _BLK)], idxb)

    def chunk(k):
        slot = k & 1
        pltpu.async_copy(x_hbm.at[idxb.at[k]], rows[slot], sg).wait()
        pltpu.async_copy(rows[slot], agg.at[idxb.at[_BLK + k]], sw[slot],
                         add=True)

    def wait_scat(slot):
        pltpu.make_async_copy(rows[slot], agg.at[idxb.at[_BLK]],
                              sw[slot]).wait()

    load_idx(0)
    for k in range(_BLK):
        if k >= 2:
            wait_scat(k & 1)
        chunk(k)

    def block(t, _):
        wait_scat(0)
        wait_scat(1)
        load_idx(t)
        for k in range(_BLK):
            if k >= 2:
                wait_scat(k & 1)
            chunk(k)
        return 0
    lax.fori_loop(1, _NBLK, block, 0)

    wait_scat(0)
    wait_scat(1)
    plsc.subcore_barrier()
    pltpu.sync_copy(agg.at[pl.ds(s * _RPS, _RPS)],
                    out_hbm.at[pl.ds(c * _NP + s * _RPS, _RPS)])


def _sc_segment_rows(x, ed, zrows):
    return pl.kernel(
        _seg_rows_body,
        out_type=jax.ShapeDtypeStruct((_NC * _NP, _H), jnp.float32),
        mesh=_sc_mesh(),
        scratch_types=[
            pltpu.VMEM((2 * _BLK, _CH), jnp.int32),
            pltpu.VMEM((_CH, _H), jnp.float32),
            pltpu.VMEM((_CH, _H), jnp.float32),
            pltpu.VMEM_SHARED((_NP, _H), jnp.float32),
        ] + [pltpu.SemaphoreType.DMA] * 3,
    )(x, ed, zrows)


def _deg_body(dst_hbm, ones_hbm, zr_hbm, out_hbm, idx_da, ones_v, agg, *sems):
    c = lax.axis_index("c")
    s = lax.axis_index("s")
    wid = s * _NC + c
    sw = sems

    pltpu.sync_copy(zr_hbm, agg.at[pl.ds(s * _RPS, _RPS)])
    pltpu.sync_copy(ones_hbm, ones_v)
    pltpu.sync_copy(dst_hbm.at[pl.ds(wid * _NB, _NB)], idx_da)
    plsc.subcore_barrier()

    def scatter(j, slot):
        pltpu.async_copy(ones_v, agg.at[idx_da.at[j]], sw[slot], add=True)

    def wait_scatter(slot):
        pltpu.make_async_copy(
            ones_v, agg.at[idx_da.at[0]], sw[slot]).wait()

    for slot in range(_SLOTS):
        scatter(slot, slot)

    def group(g, _):
        base = (g + 1) * _SLOTS
        for slot in range(_SLOTS):
            wait_scatter(slot)
            scatter(base + slot, slot)
        return 0
    lax.fori_loop(0, _NG - 1, group, 0)
    for slot in range(_SLOTS):
        wait_scatter(slot)

    plsc.subcore_barrier()
    pltpu.sync_copy(agg.at[pl.ds(s * _RPS, _RPS)],
                    out_hbm.at[pl.ds(c * _NP + s * _RPS, _RPS)])


def _sc_degrees(dst2d, ones, zrows):
    return pl.kernel(
        _deg_body,
        out_type=jax.ShapeDtypeStruct((_NC * _NP, _H), jnp.float32),
        mesh=_sc_mesh(),
        scratch_types=[
            pltpu.VMEM((_NB, _CH), jnp.int32),
            pltpu.VMEM((_CH, _H), jnp.float32),
            pltpu.VMEM_SHARED((_NP, _H), jnp.float32),
        ] + [pltpu.SemaphoreType.DMA] * _SLOTS,
    )(dst2d, ones, zrows)


def _embed_body(h_ref, w_ref, b_ref, o_ref):
    o_ref[...] = lax.dot_general(
        h_ref[...], w_ref[...], (((1,), (0,)), ((), ())),
        preferred_element_type=jnp.float32) + b_ref[...]


def _tc_embed(h, W_emb, b_emb2d):
    return pl.pallas_call(
        _embed_body,
        out_shape=jax.ShapeDtypeStruct((_N, _H), jnp.float32),
    )(h, W_emb, b_emb2d)


def _layer_body(x_ref, parts_ref, degp_ref, ws_ref, bs_ref, wn_ref, bn_ref,
                g_ref, bt_ref, o_ref):
    x = x_ref[...]
    deg = degp_ref[0:_N, 0:1] + degp_ref[_NP:_NP + _N, 0:1]
    rdeg = 1.0 / jnp.maximum(deg, 1.0)
    agg = (parts_ref[0:_N, :] + parts_ref[_NP:_NP + _N, :]) * rdeg
    out = (lax.dot_general(x, ws_ref[...], (((1,), (0,)), ((), ())),
                           preferred_element_type=jnp.float32)
           + bs_ref[...]
           + lax.dot_general(agg, wn_ref[...], (((1,), (0,)), ((), ())),
                             preferred_element_type=jnp.float32)
           + bn_ref[...])
    out = jnp.maximum(out, 0.0)
    mu = jnp.mean(out, axis=0, keepdims=True)
    var = jnp.mean((out - mu) ** 2, axis=0, keepdims=True)
    out = g_ref[...] * (out - mu) / jnp.sqrt(var + 1e-5) + bt_ref[...]
    o_ref[...] = x + out


def _tc_layer(x, parts, degp, Ws, bs2d, Wn, bn2d, g2d, bt2d):
    return pl.pallas_call(
        _layer_body,
        out_shape=jax.ShapeDtypeStruct((_N, _H), jnp.float32),
    )(x, parts, degp, Ws, bs2d, Wn, bn2d, g2d, bt2d)


def _head_body(x_ref, gid_ref, pp_ref, pn_ref, wfc_ref, o_ref):
    x = x_ref[...]
    ids = gid_ref[...]
    iota = lax.broadcasted_iota(jnp.int32, (_N, _B), 1)
    mask = (ids == iota).astype(jnp.float32)
    cnt = jnp.sum(mask, axis=0, keepdims=True)
    hgs = lax.dot_general(mask, x, (((0,), (0,)), ((), ())),
                          preferred_element_type=jnp.float32)
    hg = hgs / jnp.maximum(cnt, 1.0).reshape(_B, 1)
    cols = []
    for i in range(_P):
        dp = jnp.sum((hg - pp_ref[i:i + 1, :]) ** 2, axis=1, keepdims=True)
        cols.append(dp)
    for i in range(_P):
        dn = jnp.sum((hg - pn_ref[i:i + 1, :]) ** 2, axis=1, keepdims=True)
        cols.append(dn)
    d = jnp.concatenate(cols, axis=1)
    ss = jnp.log((d + 1.0) / (d + 1e-12))
    y = lax.dot_general(ss, wfc_ref[...], (((1,), (1,)), ((), ())),
                        preferred_element_type=jnp.float32)
    o_ref[...] = 1.0 / (1.0 + jnp.exp(-y))


def _tc_head(x, gid2d, p_pos, p_neg, W_fc):
    return pl.pallas_call(
        _head_body,
        out_shape=jax.ShapeDtypeStruct((_B, 2), jnp.float32),
    )(x, gid2d, p_pos, p_neg, W_fc)


@jax.jit
def kernel(h, edge_index, e, graph_ids, W_emb, b_emb, W_self, b_self,
           W_neigh, b_neigh, gamma, beta, p_pos, p_neg, W_fc):
    pad = _EP - _E
    src2d_pad = jnp.concatenate(
        [edge_index[0], jnp.arange(pad, dtype=jnp.int32) % _N]
    ).reshape(_NROW, _CH)
    dst2d_pad = jnp.concatenate(
        [edge_index[1],
         _N + (jnp.arange(pad, dtype=jnp.int32) % (_NP - _N))]
    ).reshape(_NROW, _CH)
    q = jnp.asarray(_EDQ)
    ed = jnp.concatenate(
        [src2d_pad[q].reshape(-1, _BLK, _CH),
         dst2d_pad[q].reshape(-1, _BLK, _CH)],
        axis=1).reshape(2 * _NROW, _CH)
    gid2d = graph_ids.reshape(_N, 1)
    zrows = jnp.zeros((_RPS, _H), jnp.float32)
    ones = jnp.ones((_CH, _H), jnp.float32)

    x = _tc_embed(h, W_emb, b_emb.reshape(1, _H))
    degp = _sc_degrees(dst2d_pad, ones, zrows)
    for l in range(3):
        parts = _sc_segment_rows(x, ed, zrows)
        x = _tc_layer(x, parts, degp,
                      W_self[l], b_self[l].reshape(1, _H),
                      W_neigh[l], b_neigh[l].reshape(1, _H),
                      gamma[l].reshape(1, _H), beta[l].reshape(1, _H))
    return _tc_head(x, gid2d, p_pos, p_neg, W_fc)


# speedup vs baseline: 3.2678x; 1.0536x over previous
import jax
import jax.numpy as jnp
import numpy as np
from jax import lax
from jax.experimental import pallas as pl
from jax.experimental.pallas import tpu as pltpu
from jax.experimental.pallas import tpu_sc as plsc

_N = 10000
_E = 320000
_H = 128
_B = 64
_P = 5

_NC = 2
_NS = 16
_NW = _NC * _NS

_CH = 128
_NCHUNK = _E // _CH
_BASE_CHUNKS = _NCHUNK // _NW
_EXTRA = _NCHUNK - _BASE_CHUNKS * _NW
_NB = 80
_EP = _NW * _NB * _CH
_NROW = _NW * _NB
_SLOTS = 4
_NG = _NB // _SLOTS
_GS = 2
_PNB = 40
_NPH = _NB // _PNB
_PNG = _PNB // _GS

_NP = 10240
_RPS = _NP // _NS

_NREAL = _E // _CH
def _make_edq():
    q = np.empty((_NROW,), dtype=np.int32)
    for w in range(_NW):
        for j in range(_NB):
            q[w * _NB + j] = w + _NW * j
    return q
_EDQ = _make_edq()


def _sc_mesh():
    return plsc.VectorSubcoreMesh(
        core_axis_name="c", subcore_axis_name="s",
        num_cores=_NC, num_subcores=_NS)


_BLK = 8
_NBLK = _NB // _BLK


def _seg_rows_body(x_hbm, ed_hbm, zr_hbm, out_hbm,
                   idxb, rows0, rows1, agg, sg, sw0, sw1):
    c = lax.axis_index("c")
    s = lax.axis_index("s")
    wid = s * _NC + c
    rows = (rows0, rows1)
    sw = (sw0, sw1)

    pltpu.sync_copy(zr_hbm, agg.at[pl.ds(s * _RPS, _RPS)])
    plsc.subcore_barrier()

    def load_idx(t):
        pltpu.sync_copy(
            ed_hbm.at[pl.ds((wid * _NBLK + t) * 2 * _BLK, 2 * _BLK)], idxb)

    def chunk(k):
        slot = k & 1
        pltpu.async_copy(x_hbm.at[idxb.at[k]], rows[slot], sg).wait()
        pltpu.async_copy(rows[slot], agg.at[idxb.at[_BLK + k]], sw[slot],
                         add=True)

    def wait_scat(slot):
        pltpu.make_async_copy(rows[slot], agg.at[idxb.at[_BLK]],
                              sw[slot]).wait()

    load_idx(0)
    for k in range(_BLK):
        if k >= 2:
            wait_scat(k & 1)
        chunk(k)

    def block(t, _):
        wait_scat(0)
        wait_scat(1)
        load_idx(t)
        for k in range(_BLK):
            if k >= 2:
                wait_scat(k & 1)
            chunk(k)
        return 0
    lax.fori_loop(1, _NBLK, block, 0)

    wait_scat(0)
    wait_scat(1)
    plsc.subcore_barrier()
    pltpu.sync_copy(agg.at[pl.ds(s * _RPS, _RPS)],
                    out_hbm.at[pl.ds(c * _NP + s * _RPS, _RPS)])


def _sc_segment_rows(x, ed, zrows):
    return pl.kernel(
        _seg_rows_body,
        out_type=jax.ShapeDtypeStruct((_NC * _NP, _H), jnp.float32),
        mesh=_sc_mesh(),
        scratch_types=[
            pltpu.VMEM((2 * _BLK, _CH), jnp.int32),
            pltpu.VMEM((_CH, _H), jnp.float32),
            pltpu.VMEM((_CH, _H), jnp.float32),
            pltpu.VMEM_SHARED((_NP, _H), jnp.float32),
        ] + [pltpu.SemaphoreType.DMA] * 3,
    )(x, ed, zrows)


def _deg_body(dst_hbm, ones_hbm, zr_hbm, out_hbm, idx_da, ones_v, agg, *sems):
    c = lax.axis_index("c")
    s = lax.axis_index("s")
    wid = s * _NC + c
    sw = sems

    pltpu.sync_copy(zr_hbm, agg.at[pl.ds(s * _RPS, _RPS)])
    pltpu.sync_copy(ones_hbm, ones_v)
    pltpu.sync_copy(dst_hbm.at[pl.ds(wid * _NB, _NB)], idx_da)
    plsc.subcore_barrier()

    def scatter(j, slot):
        pltpu.async_copy(ones_v, agg.at[idx_da.at[j]], sw[slot], add=True)

    def wait_scatter(slot):
        pltpu.make_async_copy(
            ones_v, agg.at[idx_da.at[0]], sw[slot]).wait()

    for slot in range(_SLOTS):
        scatter(slot, slot)

    def group(g, _):
        base = (g + 1) * _SLOTS
        for slot in range(_SLOTS):
            wait_scatter(slot)
            scatter(base + slot, slot)
        return 0
    lax.fori_loop(0, _NG - 1, group, 0)
    for slot in range(_SLOTS):
        wait_scatter(slot)

    plsc.subcore_barrier()
    pltpu.sync_copy(agg.at[pl.ds(s * _RPS, _RPS)],
                    out_hbm.at[pl.ds(c * _NP + s * _RPS, _RPS)])


def _sc_degrees(dst2d, ones, zrows):
    return pl.kernel(
        _deg_body,
        out_type=jax.ShapeDtypeStruct((_NC * _NP, _H), jnp.float32),
        mesh=_sc_mesh(),
        scratch_types=[
            pltpu.VMEM((_NB, _CH), jnp.int32),
            pltpu.VMEM((_CH, _H), jnp.float32),
            pltpu.VMEM_SHARED((_NP, _H), jnp.float32),
        ] + [pltpu.SemaphoreType.DMA] * _SLOTS,
    )(dst2d, ones, zrows)


def _embed_body(h_ref, w_ref, b_ref, o_ref):
    o_ref[...] = lax.dot_general(
        h_ref[...], w_ref[...], (((1,), (0,)), ((), ())),
        preferred_element_type=jnp.float32) + b_ref[...]


def _tc_embed(h, W_emb, b_emb2d):
    return pl.pallas_call(
        _embed_body,
        out_shape=jax.ShapeDtypeStruct((_N, _H), jnp.float32),
    )(h, W_emb, b_emb2d)


def _layer_body(x_ref, parts_ref, degp_ref, ws_ref, bs_ref, wn_ref, bn_ref,
                g_ref, bt_ref, o_ref):
    x = x_ref[...]
    deg = degp_ref[0:_N, 0:1] + degp_ref[_NP:_NP + _N, 0:1]
    rdeg = 1.0 / jnp.maximum(deg, 1.0)
    agg = (parts_ref[0:_N, :] + parts_ref[_NP:_NP + _N, :]) * rdeg
    out = (lax.dot_general(x, ws_ref[...], (((1,), (0,)), ((), ())),
                           preferred_element_type=jnp.float32)
           + bs_ref[...]
           + lax.dot_general(agg, wn_ref[...], (((1,), (0,)), ((), ())),
                             preferred_element_type=jnp.float32)
           + bn_ref[...])
    out = jnp.maximum(out, 0.0)
    mu = jnp.mean(out, axis=0, keepdims=True)
    var = jnp.mean((out - mu) ** 2, axis=0, keepdims=True)
    out = g_ref[...] * (out - mu) / jnp.sqrt(var + 1e-5) + bt_ref[...]
    o_ref[...] = x + out


def _tc_layer(x, parts, degp, Ws, bs2d, Wn, bn2d, g2d, bt2d):
    return pl.pallas_call(
        _layer_body,
        out_shape=jax.ShapeDtypeStruct((_N, _H), jnp.float32),
    )(x, parts, degp, Ws, bs2d, Wn, bn2d, g2d, bt2d)


def _head_body(x_ref, gid_ref, pp_ref, pn_ref, wfc_ref, o_ref):
    x = x_ref[...]
    ids = gid_ref[...]
    iota = lax.broadcasted_iota(jnp.int32, (_N, _B), 1)
    mask = (ids == iota).astype(jnp.float32)
    cnt = jnp.sum(mask, axis=0, keepdims=True)
    hgs = lax.dot_general(mask, x, (((0,), (0,)), ((), ())),
                          preferred_element_type=jnp.float32)
    hg = hgs / jnp.maximum(cnt, 1.0).reshape(_B, 1)
    cols = []
    for i in range(_P):
        dp = jnp.sum((hg - pp_ref[i:i + 1, :]) ** 2, axis=1, keepdims=True)
        cols.append(dp)
    for i in range(_P):
        dn = jnp.sum((hg - pn_ref[i:i + 1, :]) ** 2, axis=1, keepdims=True)
        cols.append(dn)
    d = jnp.concatenate(cols, axis=1)
    ss = jnp.log((d + 1.0) / (d + 1e-12))
    y = lax.dot_general(ss, wfc_ref[...], (((1,), (1,)), ((), ())),
                        preferred_element_type=jnp.float32)
    o_ref[...] = 1.0 / (1.0 + jnp.exp(-y))


def _tc_head(x, gid2d, p_pos, p_neg, W_fc):
    return pl.pallas_call(
        _head_body,
        out_shape=jax.ShapeDtypeStruct((_B, 2), jnp.float32),
    )(x, gid2d, p_pos, p_neg, W_fc)


@jax.jit
def kernel(h, edge_index, e, graph_ids, W_emb, b_emb, W_self, b_self,
           W_neigh, b_neigh, gamma, beta, p_pos, p_neg, W_fc):
    pad = _EP - _E
    src2d_pad = jnp.concatenate(
        [edge_index[0], jnp.arange(pad, dtype=jnp.int32) % _N]
    ).reshape(_NROW, _CH)
    dst2d_pad = jnp.concatenate(
        [edge_index[1],
         _N + (jnp.arange(pad, dtype=jnp.int32) % (_NP - _N))]
    ).reshape(_NROW, _CH)
    q = jnp.asarray(_EDQ)
    ed = jnp.concatenate(
        [src2d_pad[q].reshape(-1, _BLK, _CH),
         dst2d_pad[q].reshape(-1, _BLK, _CH)],
        axis=1).reshape(2 * _NROW, _CH)
    gid2d = graph_ids.reshape(_N, 1)
    zrows = jnp.zeros((_RPS, _H), jnp.float32)
    ones = jnp.ones((_CH, _H), jnp.float32)

    x = _tc_embed(h, W_emb, b_emb.reshape(1, _H))
    degp = _sc_degrees(dst2d_pad, ones, zrows)
    for l in range(3):
        parts = _sc_segment_rows(x, ed, zrows)
        x = _tc_layer(x, parts, degp,
                      W_self[l], b_self[l].reshape(1, _H),
                      W_neigh[l], b_neigh[l].reshape(1, _H),
                      gamma[l].reshape(1, _H), beta[l].reshape(1, _H))
    return _tc_head(x, gid2d, p_pos, p_neg, W_fc)


# speedup vs baseline: 3.3486x; 1.0247x over previous
import jax
import jax.numpy as jnp
import numpy as np
from jax import lax
from jax.experimental import pallas as pl
from jax.experimental.pallas import tpu as pltpu
from jax.experimental.pallas import tpu_sc as plsc

_N = 10000
_E = 320000
_H = 128
_B = 64
_P = 5

_NC = 2
_NS = 16
_NW = _NC * _NS

_CH = 128
_NCHUNK = _E // _CH
_BASE_CHUNKS = _NCHUNK // _NW
_EXTRA = _NCHUNK - _BASE_CHUNKS * _NW
_NB = 80
_EP = _NW * _NB * _CH
_NROW = _NW * _NB
_SLOTS = 4
_NG = _NB // _SLOTS
_GS = 2
_PNB = 40
_NPH = _NB // _PNB
_PNG = _PNB // _GS

_NP = 10240
_RPS = _NP // _NS

_NREAL = _E // _CH
def _make_edq():
    q = np.empty((_NROW,), dtype=np.int32)
    for w in range(_NW):
        for j in range(_NB):
            q[w * _NB + j] = w + _NW * j
    return q
_EDQ = _make_edq()


def _sc_mesh():
    return plsc.VectorSubcoreMesh(
        core_axis_name="c", subcore_axis_name="s",
        num_cores=_NC, num_subcores=_NS)


_BLK = 16
_NBLK = _NB // _BLK


def _seg_rows_body(x_hbm, ed_hbm, zr_hbm, out_hbm,
                   idxb, rows0, rows1, agg, sg, sw0, sw1):
    c = lax.axis_index("c")
    s = lax.axis_index("s")
    wid = s * _NC + c
    rows = (rows0, rows1)
    sw = (sw0, sw1)

    pltpu.sync_copy(zr_hbm, agg.at[pl.ds(s * _RPS, _RPS)])
    plsc.subcore_barrier()

    def load_idx(t):
        pltpu.sync_copy(
            ed_hbm.at[pl.ds((wid * _NBLK + t) * 2 * _BLK, 2 * _BLK)], idxb)

    def chunk(k):
        slot = k & 1
        pltpu.async_copy(x_hbm.at[idxb.at[k]], rows[slot], sg).wait()
        pltpu.async_copy(rows[slot], agg.at[idxb.at[_BLK + k]], sw[slot],
                         add=True)

    def wait_scat(slot):
        pltpu.make_async_copy(rows[slot], agg.at[idxb.at[_BLK]],
                              sw[slot]).wait()

    load_idx(0)
    for k in range(_BLK):
        if k >= 2:
            wait_scat(k & 1)
        chunk(k)

    def block(t, _):
        wait_scat(0)
        wait_scat(1)
        load_idx(t)
        for k in range(_BLK):
            if k >= 2:
                wait_scat(k & 1)
            chunk(k)
        return 0
    lax.fori_loop(1, _NBLK, block, 0)

    wait_scat(0)
    wait_scat(1)
    plsc.subcore_barrier()
    pltpu.sync_copy(agg.at[pl.ds(s * _RPS, _RPS)],
                    out_hbm.at[pl.ds(c * _NP + s * _RPS, _RPS)])


def _sc_segment_rows(x, ed, zrows):
    return pl.kernel(
        _seg_rows_body,
        out_type=jax.ShapeDtypeStruct((_NC * _NP, _H), jnp.float32),
        mesh=_sc_mesh(),
        scratch_types=[
            pltpu.VMEM((2 * _BLK, _CH), jnp.int32),
            pltpu.VMEM((_CH, _H), jnp.float32),
            pltpu.VMEM((_CH, _H), jnp.float32),
            pltpu.VMEM_SHARED((_NP, _H), jnp.float32),
        ] + [pltpu.SemaphoreType.DMA] * 3,
    )(x, ed, zrows)


def _deg_body(dst_hbm, ones_hbm, zr_hbm, out_hbm, idx_da, ones_v, agg, *sems):
    c = lax.axis_index("c")
    s = lax.axis_index("s")
    wid = s * _NC + c
    sw = sems

    pltpu.sync_copy(zr_hbm, agg.at[pl.ds(s * _RPS, _RPS)])
    pltpu.sync_copy(ones_hbm, ones_v)
    pltpu.sync_copy(dst_hbm.at[pl.ds(wid * _NB, _NB)], idx_da)
    plsc.subcore_barrier()

    def scatter(j, slot):
        pltpu.async_copy(ones_v, agg.at[idx_da.at[j]], sw[slot], add=True)

    def wait_scatter(slot):
        pltpu.make_async_copy(
            ones_v, agg.at[idx_da.at[0]], sw[slot]).wait()

    for slot in range(_SLOTS):
        scatter(slot, slot)

    def group(g, _):
        base = (g + 1) * _SLOTS
        for slot in range(_SLOTS):
            wait_scatter(slot)
            scatter(base + slot, slot)
        return 0
    lax.fori_loop(0, _NG - 1, group, 0)
    for slot in range(_SLOTS):
        wait_scatter(slot)

    plsc.subcore_barrier()
    pltpu.sync_copy(agg.at[pl.ds(s * _RPS, _RPS)],
                    out_hbm.at[pl.ds(c * _NP + s * _RPS, _RPS)])


def _sc_degrees(dst2d, ones, zrows):
    return pl.kernel(
        _deg_body,
        out_type=jax.ShapeDtypeStruct((_NC * _NP, _H), jnp.float32),
        mesh=_sc_mesh(),
        scratch_types=[
            pltpu.VMEM((_NB, _CH), jnp.int32),
            pltpu.VMEM((_CH, _H), jnp.float32),
            pltpu.VMEM_SHARED((_NP, _H), jnp.float32),
        ] + [pltpu.SemaphoreType.DMA] * _SLOTS,
    )(dst2d, ones, zrows)


def _embed_body(h_ref, w_ref, b_ref, o_ref):
    o_ref[...] = lax.dot_general(
        h_ref[...], w_ref[...], (((1,), (0,)), ((), ())),
        preferred_element_type=jnp.float32) + b_ref[...]


def _tc_embed(h, W_emb, b_emb2d):
    return pl.pallas_call(
        _embed_body,
        out_shape=jax.ShapeDtypeStruct((_N, _H), jnp.float32),
    )(h, W_emb, b_emb2d)


def _layer_body(x_ref, parts_ref, degp_ref, ws_ref, bs_ref, wn_ref, bn_ref,
                g_ref, bt_ref, o_ref):
    x = x_ref[...]
    deg = degp_ref[0:_N, 0:1] + degp_ref[_NP:_NP + _N, 0:1]
    rdeg = 1.0 / jnp.maximum(deg, 1.0)
    agg = (parts_ref[0:_N, :] + parts_ref[_NP:_NP + _N, :]) * rdeg
    out = (lax.dot_general(x, ws_ref[...], (((1,), (0,)), ((), ())),
                           preferred_element_type=jnp.float32)
           + bs_ref[...]
           + lax.dot_general(agg, wn_ref[...], (((1,), (0,)), ((), ())),
                             preferred_element_type=jnp.float32)
           + bn_ref[...])
    out = jnp.maximum(out, 0.0)
    mu = jnp.mean(out, axis=0, keepdims=True)
    var = jnp.mean((out - mu) ** 2, axis=0, keepdims=True)
    out = g_ref[...] * (out - mu) / jnp.sqrt(var + 1e-5) + bt_ref[...]
    o_ref[...] = x + out


def _tc_layer(x, parts, degp, Ws, bs2d, Wn, bn2d, g2d, bt2d):
    return pl.pallas_call(
        _layer_body,
        out_shape=jax.ShapeDtypeStruct((_N, _H), jnp.float32),
    )(x, parts, degp, Ws, bs2d, Wn, bn2d, g2d, bt2d)


def _head_body(x_ref, gid_ref, pp_ref, pn_ref, wfc_ref, o_ref):
    x = x_ref[...]
    ids = gid_ref[...]
    iota = lax.broadcasted_iota(jnp.int32, (_N, _B), 1)
    mask = (ids == iota).astype(jnp.float32)
    cnt = jnp.sum(mask, axis=0, keepdims=True)
    hgs = lax.dot_general(mask, x, (((0,), (0,)), ((), ())),
                          preferred_element_type=jnp.float32)
    hg = hgs / jnp.maximum(cnt, 1.0).reshape(_B, 1)
    cols = []
    for i in range(_P):
        dp = jnp.sum((hg - pp_ref[i:i + 1, :]) ** 2, axis=1, keepdims=True)
        cols.append(dp)
    for i in range(_P):
        dn = jnp.sum((hg - pn_ref[i:i + 1, :]) ** 2, axis=1, keepdims=True)
        cols.append(dn)
    d = jnp.concatenate(cols, axis=1)
    ss = jnp.log((d + 1.0) / (d + 1e-12))
    y = lax.dot_general(ss, wfc_ref[...], (((1,), (1,)), ((), ())),
                        preferred_element_type=jnp.float32)
    o_ref[...] = 1.0 / (1.0 + jnp.exp(-y))


def _tc_head(x, gid2d, p_pos, p_neg, W_fc):
    return pl.pallas_call(
        _head_body,
        out_shape=jax.ShapeDtypeStruct((_B, 2), jnp.float32),
    )(x, gid2d, p_pos, p_neg, W_fc)


@jax.jit
def kernel(h, edge_index, e, graph_ids, W_emb, b_emb, W_self, b_self,
           W_neigh, b_neigh, gamma, beta, p_pos, p_neg, W_fc):
    pad = _EP - _E
    src2d_pad = jnp.concatenate(
        [edge_index[0], jnp.arange(pad, dtype=jnp.int32) % _N]
    ).reshape(_NROW, _CH)
    dst2d_pad = jnp.concatenate(
        [edge_index[1],
         _N + (jnp.arange(pad, dtype=jnp.int32) % (_NP - _N))]
    ).reshape(_NROW, _CH)
    q = jnp.asarray(_EDQ)
    ed = jnp.concatenate(
        [src2d_pad[q].reshape(-1, _BLK, _CH),
         dst2d_pad[q].reshape(-1, _BLK, _CH)],
        axis=1).reshape(2 * _NROW, _CH)
    gid2d = graph_ids.reshape(_N, 1)
    zrows = jnp.zeros((_RPS, _H), jnp.float32)
    ones = jnp.ones((_CH, _H), jnp.float32)

    x = _tc_embed(h, W_emb, b_emb.reshape(1, _H))
    degp = _sc_degrees(dst2d_pad, ones, zrows)
    for l in range(3):
        parts = _sc_segment_rows(x, ed, zrows)
        x = _tc_layer(x, parts, degp,
                      W_self[l], b_self[l].reshape(1, _H),
                      W_neigh[l], b_neigh[l].reshape(1, _H),
                      gamma[l].reshape(1, _H), beta[l].reshape(1, _H))
    return _tc_head(x, gid2d, p_pos, p_neg, W_fc)


# speedup vs baseline: 3.3544x; 1.0018x over previous
import jax
import jax.numpy as jnp
import numpy as np
from jax import lax
from jax.experimental import pallas as pl
from jax.experimental.pallas import tpu as pltpu
from jax.experimental.pallas import tpu_sc as plsc

_N = 10000
_E = 320000
_H = 128
_B = 64
_P = 5

_NC = 2
_NS = 16
_NW = _NC * _NS

_CH = 128
_NB = 80
_EP = _NW * _NB * _CH
_NROW = _NW * _NB
_SLOTS = 4
_NG = _NB // _SLOTS

_NP = 10240
_RPS = _NP // _NS


def _make_edq():
    q = np.empty((_NROW,), dtype=np.int32)
    for w in range(_NW):
        for j in range(_NB):
            q[w * _NB + j] = w + _NW * j
    return q
_EDQ = _make_edq()


def _sc_mesh():
    return plsc.VectorSubcoreMesh(
        core_axis_name="c", subcore_axis_name="s",
        num_cores=_NC, num_subcores=_NS)


_BLK = 16
_NBLK = _NB // _BLK


def _seg_rows_body(x_hbm, ed_hbm, zr_hbm, out_hbm,
                   idxb, rows0, rows1, agg, sg, sw0, sw1):
    c = lax.axis_index("c")
    s = lax.axis_index("s")
    wid = s * _NC + c
    rows = (rows0, rows1)
    sw = (sw0, sw1)

    pltpu.sync_copy(zr_hbm, agg.at[pl.ds(s * _RPS, _RPS)])
    plsc.subcore_barrier()

    def load_idx(t):
        pltpu.sync_copy(
            ed_hbm.at[pl.ds((wid * _NBLK + t) * 2 * _BLK, 2 * _BLK)], idxb)

    def chunk(k):
        slot = k & 1
        pltpu.async_copy(x_hbm.at[idxb.at[k]], rows[slot], sg).wait()
        pltpu.async_copy(rows[slot], agg.at[idxb.at[_BLK + k]], sw[slot],
                         add=True)

    def wait_scat(slot):
        pltpu.make_async_copy(rows[slot], agg.at[idxb.at[_BLK]],
                              sw[slot]).wait()

    load_idx(0)
    for k in range(_BLK):
        if k >= 2:
            wait_scat(k & 1)
        chunk(k)

    def block(t, _):
        wait_scat(0)
        wait_scat(1)
        load_idx(t)
        for k in range(_BLK):
            if k >= 2:
                wait_scat(k & 1)
            chunk(k)
        return 0
    lax.fori_loop(1, _NBLK, block, 0)

    wait_scat(0)
    wait_scat(1)
    plsc.subcore_barrier()
    pltpu.sync_copy(agg.at[pl.ds(s * _RPS, _RPS)],
                    out_hbm.at[pl.ds(c * _NP + s * _RPS, _RPS)])


def _sc_segment_rows(x, ed, zrows):
    return pl.kernel(
        _seg_rows_body,
        out_type=jax.ShapeDtypeStruct((_NC * _NP, _H), jnp.float32),
        mesh=_sc_mesh(),
        scratch_types=[
            pltpu.VMEM((2 * _BLK, _CH), jnp.int32),
            pltpu.VMEM((_CH, _H), jnp.float32),
            pltpu.VMEM((_CH, _H), jnp.float32),
            pltpu.VMEM_SHARED((_NP, _H), jnp.float32),
        ] + [pltpu.SemaphoreType.DMA] * 3,
    )(x, ed, zrows)


def _deg_body(dst_hbm, ones_hbm, zr_hbm, out_hbm, idx_da, ones_v, agg, *sems):
    c = lax.axis_index("c")
    s = lax.axis_index("s")
    wid = s * _NC + c
    sw = sems

    pltpu.sync_copy(zr_hbm, agg.at[pl.ds(s * _RPS, _RPS)])
    pltpu.sync_copy(ones_hbm, ones_v)
    pltpu.sync_copy(dst_hbm.at[pl.ds(wid * _NB, _NB)], idx_da)
    plsc.subcore_barrier()

    def scatter(j, slot):
        pltpu.async_copy(ones_v, agg.at[idx_da.at[j]], sw[slot], add=True)

    def wait_scatter(slot):
        pltpu.make_async_copy(
            ones_v, agg.at[idx_da.at[0]], sw[slot]).wait()

    for slot in range(_SLOTS):
        scatter(slot, slot)

    def group(g, _):
        base = (g + 1) * _SLOTS
        for slot in range(_SLOTS):
            wait_scatter(slot)
            scatter(base + slot, slot)
        return 0
    lax.fori_loop(0, _NG - 1, group, 0)
    for slot in range(_SLOTS):
        wait_scatter(slot)

    plsc.subcore_barrier()
    pltpu.sync_copy(agg.at[pl.ds(s * _RPS, _RPS)],
                    out_hbm.at[pl.ds(c * _NP + s * _RPS, _RPS)])


def _sc_degrees(dst2d, ones, zrows):
    return pl.kernel(
        _deg_body,
        out_type=jax.ShapeDtypeStruct((_NC * _NP, _H), jnp.float32),
        mesh=_sc_mesh(),
        scratch_types=[
            pltpu.VMEM((_NB, _CH), jnp.int32),
            pltpu.VMEM((_CH, _H), jnp.float32),
            pltpu.VMEM_SHARED((_NP, _H), jnp.float32),
        ] + [pltpu.SemaphoreType.DMA] * _SLOTS,
    )(dst2d, ones, zrows)


def _embed_body(h_ref, w_ref, b_ref, o_ref):
    o_ref[...] = lax.dot_general(
        h_ref[...], w_ref[...], (((1,), (0,)), ((), ())),
        preferred_element_type=jnp.float32) + b_ref[...]


def _tc_embed(h, W_emb, b_emb2d):
    return pl.pallas_call(
        _embed_body,
        out_shape=jax.ShapeDtypeStruct((_N, _H), jnp.float32),
    )(h, W_emb, b_emb2d)


def _layer_body(x_ref, parts_ref, degp_ref, ws_ref, bs_ref, wn_ref, bn_ref,
                g_ref, bt_ref, o_ref):
    x = x_ref[...]
    deg = degp_ref[0:_N, 0:1] + degp_ref[_NP:_NP + _N, 0:1]
    rdeg = 1.0 / jnp.maximum(deg, 1.0)
    agg = (parts_ref[0:_N, :] + parts_ref[_NP:_NP + _N, :]) * rdeg
    out = (lax.dot_general(x, ws_ref[...], (((1,), (0,)), ((), ())),
                           preferred_element_type=jnp.float32)
           + bs_ref[...]
           + lax.dot_general(agg, wn_ref[...], (((1,), (0,)), ((), ())),
                             preferred_element_type=jnp.float32)
           + bn_ref[...])
    out = jnp.maximum(out, 0.0)
    mu = jnp.mean(out, axis=0, keepdims=True)
    var = jnp.mean((out - mu) ** 2, axis=0, keepdims=True)
    out = g_ref[...] * (out - mu) / jnp.sqrt(var + 1e-5) + bt_ref[...]
    o_ref[...] = x + out


def _tc_layer(x, parts, degp, Ws, bs2d, Wn, bn2d, g2d, bt2d):
    return pl.pallas_call(
        _layer_body,
        out_shape=jax.ShapeDtypeStruct((_N, _H), jnp.float32),
    )(x, parts, degp, Ws, bs2d, Wn, bn2d, g2d, bt2d)


def _head_body(x_ref, gid_ref, pp_ref, pn_ref, wfc_ref, o_ref):
    x = x_ref[...]
    ids = gid_ref[...]
    iota = lax.broadcasted_iota(jnp.int32, (_N, _B), 1)
    mask = (ids == iota).astype(jnp.float32)
    cnt = jnp.sum(mask, axis=0, keepdims=True)
    hgs = lax.dot_general(mask, x, (((0,), (0,)), ((), ())),
                          preferred_element_type=jnp.float32)
    hg = hgs / jnp.maximum(cnt, 1.0).reshape(_B, 1)
    cols = []
    for i in range(_P):
        dp = jnp.sum((hg - pp_ref[i:i + 1, :]) ** 2, axis=1, keepdims=True)
        cols.append(dp)
    for i in range(_P):
        dn = jnp.sum((hg - pn_ref[i:i + 1, :]) ** 2, axis=1, keepdims=True)
        cols.append(dn)
    d = jnp.concatenate(cols, axis=1)
    ss = jnp.log((d + 1.0) / (d + 1e-12))
    y = lax.dot_general(ss, wfc_ref[...], (((1,), (1,)), ((), ())),
                        preferred_element_type=jnp.float32)
    o_ref[...] = 1.0 / (1.0 + jnp.exp(-y))


def _tc_head(x, gid2d, p_pos, p_neg, W_fc):
    return pl.pallas_call(
        _head_body,
        out_shape=jax.ShapeDtypeStruct((_B, 2), jnp.float32),
    )(x, gid2d, p_pos, p_neg, W_fc)


@jax.jit
def kernel(h, edge_index, e, graph_ids, W_emb, b_emb, W_self, b_self,
           W_neigh, b_neigh, gamma, beta, p_pos, p_neg, W_fc):
    pad = _EP - _E
    src2d_pad = jnp.concatenate(
        [edge_index[0], jnp.arange(pad, dtype=jnp.int32) % _N]
    ).reshape(_NROW, _CH)
    dst2d_pad = jnp.concatenate(
        [edge_index[1],
         _N + (jnp.arange(pad, dtype=jnp.int32) % (_NP - _N))]
    ).reshape(_NROW, _CH)
    q = jnp.asarray(_EDQ)
    ed = jnp.concatenate(
        [src2d_pad[q].reshape(-1, _BLK, _CH),
         dst2d_pad[q].reshape(-1, _BLK, _CH)],
        axis=1).reshape(2 * _NROW, _CH)
    gid2d = graph_ids.reshape(_N, 1)
    zrows = jnp.zeros((_RPS, _H), jnp.float32)
    ones = jnp.ones((_CH, _H), jnp.float32)

    x = _tc_embed(h, W_emb, b_emb.reshape(1, _H))
    degp = _sc_degrees(dst2d_pad, ones, zrows)
    for l in range(3):
        parts = _sc_segment_rows(x, ed, zrows)
        x = _tc_layer(x, parts, degp,
                      W_self[l], b_self[l].reshape(1, _H),
                      W_neigh[l], b_neigh[l].reshape(1, _H),
                      gamma[l].reshape(1, _H), beta[l].reshape(1, _H))
    return _tc_head(x, gid2d, p_pos, p_neg, W_fc)
